# baseline (speedup 1.0000x reference)
import functools

import jax
import jax.numpy as jnp
from jax import lax
from jax.experimental import pallas as pl
from jax.experimental.pallas import tpu as pltpu
from jax.experimental.pallas import tpu_sc as plsc

BATCH = 16384
FIELDS = 26
EMBEDDING_DIM = 64

NUM_CORES = 2
NUM_SUBCORES = 16
NW = NUM_CORES * NUM_SUBCORES

B_TOTAL = BATCH * FIELDS
BLK = 128
BATCH_PER_W = BATCH // NW
CHUNKS = BATCH_PER_W // BLK
NBUF = 4

_mesh = plsc.VectorSubcoreMesh(
    core_axis_name="c", subcore_axis_name="s",
    num_cores=NUM_CORES, num_subcores=NUM_SUBCORES)

NUM_EMB = 1000000
FULL_BLOCKS = NUM_EMB // BLK
TAIL_ROWS = NUM_EMB - FULL_BLOCKS * BLK
BLOCKS_LO = FULL_BLOCKS // NW


@functools.partial(
    pl.kernel,
    out_type=jax.ShapeDtypeStruct((NUM_EMB * EMBEDDING_DIM,), jnp.float32),
    mesh=_mesh,
    scratch_types=[
        pltpu.VMEM((2 * EMBEDDING_DIM, BLK), jnp.float32),
        pltpu.VMEM((2, BLK * EMBEDDING_DIM), jnp.float32),
        pltpu.VMEM((EMBEDDING_DIM, TAIL_ROWS), jnp.float32),
        [pltpu.SemaphoreType.DMA] * 2,
        [pltpu.SemaphoreType.DMA] * 2,
    ],
    compiler_params=pltpu.CompilerParams(needs_layout_passes=False),
)
def _format_kernel(wt_hbm, out_hbm, slab, tslab, tail_v, isems, osems):
    wid = lax.axis_index("s") * NUM_CORES + lax.axis_index("c")
    lanes = lax.iota(jnp.int32, 16)
    my_blocks = BLOCKS_LO + jnp.where(wid < FULL_BLOCKS - NW * BLOCKS_LO,
                                      1, 0)

    def start_in(t, u):
        c = wid + NW * t
        pltpu.async_copy(wt_hbm.at[:, pl.ds(c * BLK, BLK)],
                         slab.at[pl.ds(u * EMBEDDING_DIM, EMBEDDING_DIM)],
                         isems[u])

    def wait_in(u):
        pltpu.make_async_copy(
            wt_hbm.at[:, pl.ds(0, BLK)],
            slab.at[pl.ds(u * EMBEDDING_DIM, EMBEDDING_DIM)],
            isems[u]).wait()

    def out_slice(t):
        c = wid + NW * t
        return out_hbm.at[pl.ds(c * BLK * EMBEDDING_DIM,
                                BLK * EMBEDDING_DIM)]

    def wait_out(u):
        pltpu.make_async_copy(tslab.at[u], out_slice(0), osems[u]).wait()

    def do_block(t, u, first_round):
        wait_in(u)
        if not first_round:
            wait_out(u)

        @plsc.parallel_loop(0, EMBEDDING_DIM, unroll=8)
        def _row(j):
            for q in range(BLK // 16):
                vals = slab[u * EMBEDDING_DIM + j, pl.ds(16 * q, 16)]
                addrs = (lanes + 16 * q) * EMBEDDING_DIM + j
                plsc.store_scatter(tslab, [lanes * 0 + u, addrs], vals)

        @pl.when(t + 2 < my_blocks)
        def _():
            start_in(t + 2, u)

        pltpu.async_copy(tslab.at[u], out_slice(t), osems[u])

    start_in(0, 0)
    start_in(1, 1)
    do_block(0, 0, True)
    do_block(1, 1, True)

    @pl.loop(1, BLOCKS_LO // 2)
    def _j(j):
        do_block(2 * j, 0, False)
        do_block(2 * j + 1, 1, False)

    @pl.when(my_blocks > BLOCKS_LO)
    def _():
        do_block(BLOCKS_LO, 0, False)

    wait_out(0)
    wait_out(1)

    @pl.when(wid == NW - 1)
    def _tail():
        pltpu.sync_copy(wt_hbm.at[:, pl.ds(FULL_BLOCKS * BLK, TAIL_ROWS)],
                        tail_v)

        @plsc.parallel_loop(0, EMBEDDING_DIM, unroll=8)
        def _row(j):
            for q in range(TAIL_ROWS // 16):
                vals = tail_v[j, pl.ds(16 * q, 16)]
                addrs = (lanes + 16 * q) * EMBEDDING_DIM + j
                plsc.store_scatter(tslab, [lanes * 0, addrs], vals)

        pltpu.sync_copy(
            tslab.at[0, pl.ds(0, TAIL_ROWS * EMBEDDING_DIM)],
            out_hbm.at[pl.ds(FULL_BLOCKS * BLK * EMBEDDING_DIM,
                             TAIL_ROWS * EMBEDDING_DIM)])


@functools.partial(
    pl.kernel,
    out_type=jax.ShapeDtypeStruct((B_TOTAL,), jnp.int32),
    mesh=_mesh,
    scratch_types=[pltpu.VMEM((8, BATCH_PER_W), jnp.int32)],
)
def _decode_kernel(xt_hbm, out_hbm, vm):
    wid = lax.axis_index("s") * NUM_CORES + lax.axis_index("c")
    col = wid * BATCH_PER_W
    for r in range((FIELDS + 7) // 8):
        nrows = min(8, FIELDS - 8 * r)
        pltpu.sync_copy(
            xt_hbm.at[pl.ds(8 * r, nrows), pl.ds(col, BATCH_PER_W)],
            vm.at[pl.ds(0, nrows)])
        for s in range(nrows):
            f = 8 * r + s
            pltpu.sync_copy(
                vm.at[s],
                out_hbm.at[pl.ds(f * BATCH + col, BATCH_PER_W)])


@functools.partial(
    pl.kernel,
    out_type=jax.ShapeDtypeStruct((BATCH, FIELDS * EMBEDDING_DIM),
                                  jnp.float32),
    mesh=_mesh,
    scratch_types=[
        pltpu.VMEM((FIELDS, BATCH_PER_W), jnp.int32),
        pltpu.VMEM((NBUF, BLK, EMBEDDING_DIM), jnp.float32),
        [pltpu.SemaphoreType.DMA] * NBUF,
    ],
    compiler_params=pltpu.CompilerParams(use_tc_tiling_on_sc=False,
                                         needs_layout_passes=False),
)
def _gather_kernel(idx_hbm, table_hbm, out_hbm, idx_v, rows_v, sems):
    wid = lax.axis_index("s") * NUM_CORES + lax.axis_index("c")
    col = wid * BATCH_PER_W
    for f in range(FIELDS):
        pltpu.sync_copy(idx_hbm.at[pl.ds(f * BATCH + col, BATCH_PER_W)],
                        idx_v.at[f])

    def start_gather(g, b):
        f, cc = g // CHUNKS, g % CHUNKS
        pltpu.async_copy(table_hbm.at[idx_v.at[f, pl.ds(cc * BLK, BLK)]],
                         rows_v.at[b], sems[b])

    def wait_gather(b):
        pltpu.make_async_copy(table_hbm.at[idx_v.at[0, pl.ds(0, BLK)]],
                              rows_v.at[b], sems[b]).wait()

    for b in range(NBUF - 1):
        start_gather(b, b)

    @pl.loop(0, FIELDS)
    def _body(f):
        for cc in range(CHUNKS):
            g = f * CHUNKS + cc
            wait_gather(cc)
            gnext = g + NBUF - 1

            @pl.when(gnext < FIELDS * CHUNKS)
            def _():
                start_gather(gnext, (cc + NBUF - 1) % NBUF)

            pltpu.sync_copy(
                rows_v.at[cc],
                out_hbm.at[pl.ds(col + cc * BLK, BLK),
                           pl.ds(f * EMBEDDING_DIM, EMBEDDING_DIM)])


def kernel(x, weight):
    table1d = _format_kernel(weight.T)
    idx1d = _decode_kernel(x.T)
    out = _gather_kernel(idx1d, table1d.reshape(NUM_EMB, EMBEDDING_DIM))
    return out.reshape(BATCH, FIELDS, EMBEDDING_DIM)


# speedup vs baseline: 1.3973x; 1.3973x over previous
import functools

import jax
import jax.numpy as jnp
from jax import lax
from jax.experimental import pallas as pl
from jax.experimental.pallas import tpu as pltpu
from jax.experimental.pallas import tpu_sc as plsc

BATCH = 16384
FIELDS = 26
EMBEDDING_DIM = 64

NUM_CORES = 2
NUM_SUBCORES = 16
NW = NUM_CORES * NUM_SUBCORES

B_TOTAL = BATCH * FIELDS
BLK = 128
BATCH_PER_W = BATCH // NW
CHUNKS = BATCH_PER_W // BLK
NBUF = 4

_mesh = plsc.VectorSubcoreMesh(
    core_axis_name="c", subcore_axis_name="s",
    num_cores=NUM_CORES, num_subcores=NUM_SUBCORES)

@functools.partial(
    pl.kernel,
    out_type=jax.ShapeDtypeStruct((B_TOTAL,), jnp.int32),
    mesh=_mesh,
    scratch_types=[pltpu.VMEM((8, BATCH_PER_W), jnp.int32)],
)
def _decode_kernel(xt_hbm, out_hbm, vm):
    wid = lax.axis_index("s") * NUM_CORES + lax.axis_index("c")
    col = wid * BATCH_PER_W
    for r in range((FIELDS + 7) // 8):
        nrows = min(8, FIELDS - 8 * r)
        pltpu.sync_copy(
            xt_hbm.at[pl.ds(8 * r, nrows), pl.ds(col, BATCH_PER_W)],
            vm.at[pl.ds(0, nrows)])
        for s in range(nrows):
            f = 8 * r + s
            pltpu.sync_copy(
                vm.at[s],
                out_hbm.at[pl.ds(f * BATCH + col, BATCH_PER_W)])


@functools.partial(
    pl.kernel,
    out_type=jax.ShapeDtypeStruct((BATCH, FIELDS * EMBEDDING_DIM),
                                  jnp.float32),
    mesh=_mesh,
    scratch_types=[
        pltpu.VMEM((FIELDS, BATCH_PER_W), jnp.int32),
        pltpu.VMEM((NBUF, BLK, EMBEDDING_DIM), jnp.float32),
        [pltpu.SemaphoreType.DMA] * NBUF,
    ],
    compiler_params=pltpu.CompilerParams(use_tc_tiling_on_sc=False,
                                         needs_layout_passes=False),
)
def _gather_kernel(idx_hbm, table_hbm, out_hbm, idx_v, rows_v, sems):
    wid = lax.axis_index("s") * NUM_CORES + lax.axis_index("c")
    col = wid * BATCH_PER_W
    for f in range(FIELDS):
        pltpu.sync_copy(idx_hbm.at[pl.ds(f * BATCH + col, BATCH_PER_W)],
                        idx_v.at[f])

    def start_gather(g, b):
        f, cc = g // CHUNKS, g % CHUNKS
        pltpu.async_copy(table_hbm.at[idx_v.at[f, pl.ds(cc * BLK, BLK)]],
                         rows_v.at[b], sems[b])

    def wait_gather(b):
        pltpu.make_async_copy(table_hbm.at[idx_v.at[0, pl.ds(0, BLK)]],
                              rows_v.at[b], sems[b]).wait()

    for b in range(NBUF - 1):
        start_gather(b, b)

    @pl.loop(0, FIELDS)
    def _body(f):
        for cc in range(CHUNKS):
            g = f * CHUNKS + cc
            wait_gather(cc)
            gnext = g + NBUF - 1

            @pl.when(gnext < FIELDS * CHUNKS)
            def _():
                start_gather(gnext, (cc + NBUF - 1) % NBUF)

            pltpu.sync_copy(
                rows_v.at[cc],
                out_hbm.at[pl.ds(col + cc * BLK, BLK),
                           pl.ds(f * EMBEDDING_DIM, EMBEDDING_DIM)])


def kernel(x, weight):
    idx1d = _decode_kernel(x.T)
    out = _gather_kernel(idx1d, weight)
    return out.reshape(BATCH, FIELDS, EMBEDDING_DIM)
